# baseline (speedup 1.0000x reference)
import functools

import numpy as np

import jax
import jax.numpy as jnp
from jax import lax
from jax.experimental import pallas as pl
from jax.experimental.pallas import tpu as pltpu
from jax.experimental.pallas import tpu_sc as plsc

_ROWS, _COLS = 16384, 1024
_SC_ROWS = 10240
_TC_ROWS = _ROWS - _SC_ROWS
_NC, _NS = 2, 16
_NW = _NC * _NS
_ROWS_W = _SC_ROWS // _NW
_CH_ROWS = 16
_CHUNK = _CH_ROWS * _COLS
_NCHUNK = _ROWS_W // _CH_ROWS
_TC_BLK = 256
_LANES = 16
_UNROLL = 8
_BOUNDS = [-3.0, -2.8, -2.6, -2.4, -2.2, -2.0, -1.8, -1.6, -1.4, -1.2,
           -1.0, -0.8, -0.6, -0.4, -0.2, 0.0, 0.2, 0.4, 0.6, 0.8, 1.0,
           1.2, 1.4, 1.6, 1.8, 2.0, 2.2, 2.4, 2.6, 2.8, 3.0, float("inf")]

_SCALE = np.float32(5.0)
_SHIFT = np.float32(15.9999)
_ZERO = np.float32(0.0)
_TOPF = np.float32(31.0)


def _bucketize_vec(v, tab_v):
    t = v * _SCALE + _SHIFT
    t = jnp.minimum(jnp.maximum(t, _ZERO), _TOPF)
    c = t.astype(jnp.int32)
    g = plsc.load_gather(tab_v, [c])
    return c + (v >= g).astype(jnp.int32)


def _compute_chunk(in_v, out_v, b, tab_v):
    @plsc.parallel_loop(0, _CHUNK, _LANES, unroll=_UNROLL)
    def vec_body(o):
        r = o >> 10
        col = o & (_COLS - 1)
        v = in_v[b, r, pl.ds(col, _LANES)]
        out_v[b, r, pl.ds(col, _LANES)] = _bucketize_vec(v, tab_v)


def _body(x_hbm, tab_hbm, out_hbm, in_v, out_v, tab_v,
          isem0, isem1, osem0, osem1):
    wid = lax.axis_index("s") * _NC + lax.axis_index("c")
    row0 = wid * _ROWS_W
    isem = (isem0, isem1)
    osem = (osem0, osem1)

    def in_slice(g):
        return x_hbm.at[pl.ds(row0 + g * _CH_ROWS, _CH_ROWS), :]

    def out_slice(g):
        return out_hbm.at[pl.ds(row0 + g * _CH_ROWS, _CH_ROWS), :]

    for b in (0, 1):
        pltpu.async_copy(in_slice(b), in_v.at[b], isem[b])
    pltpu.sync_copy(tab_hbm, tab_v)

    for b in (0, 1):
        pltpu.make_async_copy(in_slice(b), in_v.at[b], isem[b]).wait()
        _compute_chunk(in_v, out_v, b, tab_v)
        pltpu.async_copy(out_v.at[b], out_slice(b), osem[b])
        pltpu.async_copy(in_slice(b + 2), in_v.at[b], isem[b])

    def pair(p, _):
        for b in (0, 1):
            g = 2 * p + b
            pltpu.make_async_copy(in_slice(g), in_v.at[b], isem[b]).wait()
            pltpu.make_async_copy(out_v.at[b], out_slice(g), osem[b]).wait()
            _compute_chunk(in_v, out_v, b, tab_v)
            pltpu.async_copy(out_v.at[b], out_slice(g), osem[b])
            pltpu.async_copy(in_slice(g + 2), in_v.at[b], isem[b])
        return 0

    lax.fori_loop(1, _NCHUNK // 2 - 1, pair, 0)

    for b in (0, 1):
        g = _NCHUNK - 2 + b
        pltpu.make_async_copy(in_slice(g), in_v.at[b], isem[b]).wait()
        pltpu.make_async_copy(out_v.at[b], out_slice(g), osem[b]).wait()
        _compute_chunk(in_v, out_v, b, tab_v)
        pltpu.async_copy(out_v.at[b], out_slice(g), osem[b])
    for b in (0, 1):
        g = _NCHUNK - 2 + b
        pltpu.make_async_copy(out_v.at[b], out_slice(g), osem[b]).wait()


def _tc_body(x_ref, out_ref):
    v = x_ref[...]
    acc = (v >= np.float32(_BOUNDS[0])).astype(jnp.int32)
    for bval in _BOUNDS[1:31]:
        acc += (v >= np.float32(bval)).astype(jnp.int32)
    out_ref[...] = acc


@functools.partial(jax.jit)
def _run(x, table):
    mesh = plsc.VectorSubcoreMesh(core_axis_name="c", subcore_axis_name="s")
    sc_ker = functools.partial(
        pl.kernel,
        mesh=mesh,
        out_type=jax.ShapeDtypeStruct((_SC_ROWS, _COLS), jnp.int32),
        scratch_types=[
            pltpu.VMEM((2, _CH_ROWS, _COLS), jnp.float32),
            pltpu.VMEM((2, _CH_ROWS, _COLS), jnp.int32),
            pltpu.VMEM((len(_BOUNDS),), jnp.float32),
            pltpu.SemaphoreType.DMA,
            pltpu.SemaphoreType.DMA,
            pltpu.SemaphoreType.DMA,
            pltpu.SemaphoreType.DMA,
        ],
        compiler_params=pltpu.CompilerParams(needs_layout_passes=False),
    )(_body)
    out_sc = sc_ker(x, table)

    tc_off = _SC_ROWS // _TC_BLK
    out_tc = pl.pallas_call(
        _tc_body,
        grid=(_TC_ROWS // _TC_BLK,),
        in_specs=[pl.BlockSpec((_TC_BLK, _COLS), lambda i: (i + tc_off, 0))],
        out_specs=pl.BlockSpec((_TC_BLK, _COLS), lambda i: (i, 0)),
        out_shape=jax.ShapeDtypeStruct((_TC_ROWS, _COLS), jnp.int32),
    )(x)
    return jnp.concatenate([out_sc, out_tc], axis=0)


def kernel(x):
    table = np.asarray(_BOUNDS, dtype=np.float32)
    return _run(x, table)


# speedup vs baseline: 1.6326x; 1.6326x over previous
import functools

import numpy as np

import jax
import jax.numpy as jnp
from jax import lax
from jax.experimental import pallas as pl
from jax.experimental.pallas import tpu as pltpu
from jax.experimental.pallas import tpu_sc as plsc

_ROWS, _COLS = 16384, 1024
_NC, _NS = 2, 16
_NW = _NC * _NS
_ROWS_W = _ROWS // _NW
_CH_ROWS = 16
_CHUNK = _CH_ROWS * _COLS
_NCHUNK = _ROWS_W // _CH_ROWS
_LANES = 16
_UNROLL = 8
_BOUNDS = [-3.0, -2.8, -2.6, -2.4, -2.2, -2.0, -1.8, -1.6, -1.4, -1.2,
           -1.0, -0.8, -0.6, -0.4, -0.2, 0.0, 0.2, 0.4, 0.6, 0.8, 1.0,
           1.2, 1.4, 1.6, 1.8, 2.0, 2.2, 2.4, 2.6, 2.8, 3.0, float("inf")]

_SCALE = np.float32(5.0)
_SHIFT = np.float32(15.9999)
_ZERO = np.float32(0.0)
_TOPF = np.float32(31.0)


def _bucketize_vec(v, tab_v):
    t = v * _SCALE + _SHIFT
    t = jnp.minimum(jnp.maximum(t, _ZERO), _TOPF)
    c = t.astype(jnp.int32)
    g = plsc.load_gather(tab_v, [c])
    return c + (v >= g).astype(jnp.int32)


def _compute_chunk(in_v, out_v, b, tab_v):
    @plsc.parallel_loop(0, _CHUNK, _LANES, unroll=_UNROLL)
    def vec_body(o):
        r = o >> 10
        col = o & (_COLS - 1)
        v = in_v[b, r, pl.ds(col, _LANES)]
        out_v[b, r, pl.ds(col, _LANES)] = _bucketize_vec(v, tab_v)


def _body(x_hbm, tab_hbm, out_hbm, in_v, out_v, tab_v,
          isem0, isem1, osem0, osem1):
    wid = lax.axis_index("s") * _NC + lax.axis_index("c")
    row0 = wid * _ROWS_W
    isem = (isem0, isem1)
    osem = (osem0, osem1)

    def in_slice(g):
        return x_hbm.at[pl.ds(row0 + g * _CH_ROWS, _CH_ROWS), :]

    def out_slice(g):
        return out_hbm.at[pl.ds(row0 + g * _CH_ROWS, _CH_ROWS), :]

    for b in (0, 1):
        pltpu.async_copy(in_slice(b), in_v.at[b], isem[b])
    pltpu.sync_copy(tab_hbm, tab_v)

    for b in (0, 1):
        pltpu.make_async_copy(in_slice(b), in_v.at[b], isem[b]).wait()
        _compute_chunk(in_v, out_v, b, tab_v)
        pltpu.async_copy(out_v.at[b], out_slice(b), osem[b])
        pltpu.async_copy(in_slice(b + 2), in_v.at[b], isem[b])

    def pair(p, _):
        for b in (0, 1):
            g = 2 * p + b
            pltpu.make_async_copy(in_slice(g), in_v.at[b], isem[b]).wait()
            pltpu.make_async_copy(out_v.at[b], out_slice(g), osem[b]).wait()
            _compute_chunk(in_v, out_v, b, tab_v)
            pltpu.async_copy(out_v.at[b], out_slice(g), osem[b])
            pltpu.async_copy(in_slice(g + 2), in_v.at[b], isem[b])
        return 0

    lax.fori_loop(1, _NCHUNK // 2 - 1, pair, 0)

    for b in (0, 1):
        g = _NCHUNK - 2 + b
        pltpu.make_async_copy(in_slice(g), in_v.at[b], isem[b]).wait()
        pltpu.make_async_copy(out_v.at[b], out_slice(g), osem[b]).wait()
        _compute_chunk(in_v, out_v, b, tab_v)
        pltpu.async_copy(out_v.at[b], out_slice(g), osem[b])
    for b in (0, 1):
        g = _NCHUNK - 2 + b
        pltpu.make_async_copy(out_v.at[b], out_slice(g), osem[b]).wait()


@functools.partial(jax.jit)
def _run(x, table):
    mesh = plsc.VectorSubcoreMesh(core_axis_name="c", subcore_axis_name="s")
    ker = functools.partial(
        pl.kernel,
        mesh=mesh,
        out_type=jax.ShapeDtypeStruct((_ROWS, _COLS), jnp.int32),
        scratch_types=[
            pltpu.VMEM((2, _CH_ROWS, _COLS), jnp.float32),
            pltpu.VMEM((2, _CH_ROWS, _COLS), jnp.int32),
            pltpu.VMEM((len(_BOUNDS),), jnp.float32),
            pltpu.SemaphoreType.DMA,
            pltpu.SemaphoreType.DMA,
            pltpu.SemaphoreType.DMA,
            pltpu.SemaphoreType.DMA,
        ],
        compiler_params=pltpu.CompilerParams(needs_layout_passes=False),
    )(_body)
    return ker(x, table)


def kernel(x):
    table = np.asarray(_BOUNDS, dtype=np.float32)
    return _run(x, table)
